# baseline (speedup 1.0000x reference)
import functools

import jax
import jax.numpy as jnp
from jax.experimental import pallas as pl

_N = 10000
_E = 320000
_EMBED = 64
_NUM_LAYERS = 2
_INT_MIN = jnp.iinfo(jnp.int32).min


def _mm_body(a_ref, b_ref, o_ref):
    o_ref[...] = jnp.dot(a_ref[...], b_ref[...],
                         preferred_element_type=jnp.float32)


@jax.jit
def _mm(a, b):
    return pl.pallas_call(
        _mm_body,
        out_shape=jax.ShapeDtypeStruct((a.shape[0], b.shape[1]), jnp.float32),
    )(a, b)


def _match_rounds(src, dst, valid, score):
    eid = jnp.arange(_E, dtype=jnp.int32)
    sbits = jax.lax.bitcast_convert_type(score, jnp.int32)
    node_ids = jnp.arange(_N, dtype=jnp.int32)

    def cond(state):
        active, _, _, _ = state
        return jnp.any(active)

    def body(state):
        active, matched, cluster, pair_score = state
        sb = jnp.where(active, sbits, _INT_MIN)
        best = jnp.full((_N,), _INT_MIN, jnp.int32)
        best = best.at[src].max(sb).at[dst].max(sb)
        cand_s = jnp.where(active & (sbits == best[src]), eid, _E)
        cand_d = jnp.where(active & (sbits == best[dst]), eid, _E)
        bidx = jnp.full((_N,), _E, jnp.int32)
        bidx = bidx.at[src].min(cand_s).at[dst].min(cand_d)
        win = active & (bidx[src] == eid) & (bidx[dst] == eid)
        lab = jnp.minimum(src, dst)
        sidx = jnp.where(win, src, _N)
        didx = jnp.where(win, dst, _N)
        lidx = jnp.where(win, lab, _N)
        matched = matched.at[sidx].set(True, mode="drop")
        matched = matched.at[didx].set(True, mode="drop")
        cluster = cluster.at[sidx].set(lab, mode="drop")
        cluster = cluster.at[didx].set(lab, mode="drop")
        pair_score = pair_score.at[lidx].set(score, mode="drop")
        active = active & ~matched[src] & ~matched[dst]
        return active, matched, cluster, pair_score

    state = (
        valid,
        jnp.zeros((_N,), bool),
        node_ids,
        jnp.ones((_N,), jnp.float32),
    )
    _, _, cluster, pair_score = jax.lax.while_loop(cond, body, state)
    return cluster, pair_score


@jax.jit
def _pipeline(x, edge_index, W_embed, lin_w, lin_b, fc_w, fc_b):
    src = edge_index[0].astype(jnp.int32)
    dst = edge_index[1].astype(jnp.int32)
    valid = jnp.ones((_E,), bool)
    z = _mm(x, W_embed)
    emb = [jnp.sum(z, axis=0)]
    for l in range(_NUM_LAYERS):
        w = lin_w[l].reshape(2, _EMBED).T
        ab = _mm(z, w)
        a = ab[:, 0]
        b = ab[:, 1]
        raw = a[src] + b[dst] + lin_b[l]
        raw = jnp.where(valid, raw, -jnp.inf)
        m = jnp.full((_N,), -jnp.inf, jnp.float32).at[dst].max(raw)
        m = jnp.where(jnp.isfinite(m), m, 0.0)
        ex = jnp.where(valid, jnp.exp(raw - m[dst]), 0.0)
        ssum = jnp.zeros((_N,), jnp.float32).at[dst].add(ex)
        score = ex / (ssum[dst] + 1e-16) + 0.5

        cluster, pair_score = _match_rounds(src, dst, valid, score)

        z = jnp.zeros((_N, _EMBED), jnp.float32).at[cluster].add(z)
        z = z * pair_score[:, None]

        key = jnp.where(valid, cluster[src] * _N + cluster[dst], _N * _N)
        eid = jnp.arange(_E, dtype=jnp.int32)
        table = jnp.zeros((_N * _N + 8,), jnp.int32).at[key].set(eid)
        valid = valid & (table[key] == eid)
        src = jnp.where(valid, key // _N, 0).astype(jnp.int32)
        dst = jnp.where(valid, key % _N, 0).astype(jnp.int32)

        emb.append(jnp.sum(z, axis=0))
    Z = jnp.concatenate(emb).reshape(1, -1)
    return _mm(Z, fc_w) + fc_b


def kernel(x, edge_index, batch, W_embed, lin_w, lin_b, fc_w, fc_b):
    del batch
    return _pipeline(x, edge_index, W_embed, lin_w, lin_b, fc_w, fc_b)
